# baseline (speedup 1.0000x reference)
import functools

import jax
import jax.numpy as jnp
from jax import lax
from jax.experimental import pallas as pl
from jax.experimental.pallas import tpu as pltpu
from jax.experimental.pallas import tpu_sc as plsc

_D = 64
_ROWS = 4096
_COLS = 200
_NC, _NS = 2, 16
_NW = _NC * _NS
_ROWS_W = _ROWS // _NW
_NBUF = 4
_STEPS = _COLS // _NBUF


def _sc_embedding_gather(tokens_t, table):
  mesh = plsc.VectorSubcoreMesh(core_axis_name="c", subcore_axis_name="s")

  @functools.partial(
      pl.kernel,
      mesh=mesh,
      compiler_params=pltpu.CompilerParams(
          use_tc_tiling_on_sc=False, needs_layout_passes=False
      ),
      out_type=jax.ShapeDtypeStruct(
          (_COLS, _D // 8, _ROWS // _ROWS_W, 8, _ROWS_W), jnp.float32
      ),
      scratch_types=[
          pltpu.VMEM((_COLS, _ROWS_W), jnp.int32),
          [pltpu.VMEM((_ROWS_W, _D), jnp.float32) for _ in range(_NBUF)],
          [pltpu.VMEM((_D // 8, 8, _ROWS_W), jnp.float32)
           for _ in range(_NBUF)],
          pltpu.SemaphoreType.DMA,
          [pltpu.SemaphoreType.DMA for _ in range(_NBUF)],
          [pltpu.SemaphoreType.DMA for _ in range(_NBUF)],
      ],
  )
  def emb_kernel(tok_hbm, tab_hbm, out_hbm, blk, grows, sbufs, isem, gsems,
                 ssems):
    wid = lax.axis_index("s") * _NC + lax.axis_index("c")
    row0 = wid * _ROWS_W

    blk_cp = pltpu.make_async_copy(
        tok_hbm.at[:, pl.ds(row0, _ROWS_W)], blk, isem
    )
    blk_cp.start()
    blk_cp.wait()

    def gather(c, b):
      return pltpu.make_async_copy(
          tab_hbm.at[blk.at[c, pl.ds(0, _ROWS_W)]], grows[b], gsems[b]
      )

    def stores(c, b):
      return [
          pltpu.make_async_copy(
              sbufs[b].at[ft], out_hbm.at[c, ft, wid], ssems[b]
          )
          for ft in range(_D // 8)
      ]

    lane_rows = [lax.iota(jnp.int32, 16) + 16 * k for k in range(8)]

    def transpose_unit(b):
      def ff_body(ff, carry):
        fsplat = jnp.full((16,), ff, jnp.int32)
        for k in range(8):
          v = plsc.load_gather(grows[b], [lane_rows[k], fsplat])
          sbufs[b][ff // 8, ff % 8, pl.ds(16 * k, 16)] = v
        return carry

      lax.fori_loop(0, _D, ff_body, 0)

    for b in range(_NBUF):
      gather(b, b).start()

    def body(i, carry):
      for b in range(_NBUF):
        c = i * _NBUF + b
        gather(c, b).wait()

        @pl.when(c >= _NBUF)
        def _():
          for st in stores(c - _NBUF, b):
            st.wait()

        transpose_unit(b)
        for st in stores(c, b):
          st.start()

        @pl.when(c + _NBUF < _COLS)
        def _():
          gather(c + _NBUF, b).start()

      return carry

    lax.fori_loop(0, _STEPS, body, 0)
    for b in range(_NBUF):
      for st in stores(_COLS - _NBUF + b, b):
        st.wait()

  return emb_kernel(tokens_t, table)


def kernel(tokens, embedding):
  out5 = _sc_embedding_gather(tokens.T.astype(jnp.int32), embedding)
  return out5.transpose(2, 4, 0, 1, 3).reshape(_ROWS, _COLS, _D)


# speedup vs baseline: 1.9329x; 1.9329x over previous
import functools

import jax
import jax.numpy as jnp
from jax import lax
from jax.experimental import pallas as pl
from jax.experimental.pallas import tpu as pltpu
from jax.experimental.pallas import tpu_sc as plsc

_D = 64
_DP = 128
_ROWS = 4096
_COLS = 200
_COLS_PAD = 208
_NC, _NS = 2, 16
_NW = _NC * _NS
_ROWS_W = _ROWS // _NW
_SPLITS = ((0, 104), (104, 96))
_NBUF = 4
_LOOK = 2
_STEPS = _ROWS_W // _NBUF


def _sc_embedding_gather(tokens_t, table):
  mesh = plsc.VectorSubcoreMesh(core_axis_name="c", subcore_axis_name="s")

  @functools.partial(
      pl.kernel,
      mesh=mesh,
      compiler_params=pltpu.CompilerParams(
          use_tc_tiling_on_sc=False, needs_layout_passes=False
      ),
      out_type=jax.ShapeDtypeStruct((_ROWS * _COLS, _DP), jnp.float32),
      scratch_types=[
          pltpu.VMEM((_COLS_PAD, _ROWS_W), jnp.int32),
          pltpu.VMEM((_ROWS_W, _COLS_PAD), jnp.int32),
          [pltpu.VMEM((_COLS, _D), jnp.float32) for _ in range(_NBUF)],
          pltpu.SemaphoreType.DMA,
          [pltpu.SemaphoreType.DMA for _ in range(_NBUF)],
          [pltpu.SemaphoreType.DMA for _ in range(_NBUF)],
      ],
  )
  def emb_kernel(tok_hbm, tab_hbm, out_hbm, blk, idxt, rows, isem, gsems,
                 ssems):
    wid = lax.axis_index("s") * _NC + lax.axis_index("c")
    row0 = wid * _ROWS_W

    blk_cp = pltpu.make_async_copy(
        tok_hbm.at[:, pl.ds(row0, _ROWS_W)], blk.at[pl.ds(0, _COLS)], isem
    )
    blk_cp.start()
    blk_cp.wait()

    lanes = lax.iota(jnp.int32, 16)

    def trans_body(r, carry):
      rsplat = jnp.full((16,), r, jnp.int32)
      for cb in range(_COLS_PAD // 16):
        v = plsc.load_gather(blk, [lanes + cb * 16, rsplat])
        idxt[r, pl.ds(cb * 16, 16)] = v
      return carry

    lax.fori_loop(0, _ROWS_W, trans_body, 0)

    def gathers(r, b):
      return [
          pltpu.make_async_copy(
              tab_hbm.at[idxt.at[r, pl.ds(off, sz)]],
              rows[b].at[pl.ds(off, sz)],
              gsems[b],
          )
          for off, sz in _SPLITS
      ]

    def store(r, b):
      return pltpu.make_async_copy(
          rows[b],
          out_hbm.at[pl.ds((row0 + r) * _COLS, _COLS), pl.ds(0, _D)],
          ssems[b],
      )

    for b in range(_LOOK):
      for g in gathers(b, b):
        g.start()

    def body(i, carry):
      for b in range(_NBUF):
        r = i * _NBUF + b
        for g in gathers(r, b):
          g.wait()
        store(r, b).start()

        t = r + _LOOK
        bt = (b + _LOOK) % _NBUF

        @pl.when(t < _ROWS_W)
        def _():
          @pl.when(t >= _NBUF)
          def _():
            store(t - _NBUF, bt).wait()

          for g in gathers(t, bt):
            g.start()

      return carry

    lax.fori_loop(0, _STEPS, body, 0)
    for b in range(_NBUF):
      store(_ROWS_W - _NBUF + b, b).wait()

  return emb_kernel(tokens_t, table)


def kernel(tokens, embedding):
  out2 = _sc_embedding_gather(tokens.T.astype(jnp.int32), embedding)
  return out2[:, :_D].reshape(_ROWS, _COLS, _D)


# speedup vs baseline: 1.9603x; 1.0142x over previous
import functools

import jax
import jax.numpy as jnp
from jax import lax
from jax.experimental import pallas as pl
from jax.experimental.pallas import tpu as pltpu
from jax.experimental.pallas import tpu_sc as plsc

_D = 64
_DP = 128
_ROWS = 4096
_COLS = 200
_COLS_PAD = 208
_NC, _NS = 2, 16
_NW = _NC * _NS
_ROWS_W = _ROWS // _NW
_SPLITS = ((0, 104), (104, 96))
_NBUF = 4
_LOOK = 2
_STEPS = _ROWS_W // _NBUF


def _sc_embedding_gather(tokens_t, table):
  mesh = plsc.VectorSubcoreMesh(core_axis_name="c", subcore_axis_name="s")

  @functools.partial(
      pl.kernel,
      mesh=mesh,
      compiler_params=pltpu.CompilerParams(
          use_tc_tiling_on_sc=False, needs_layout_passes=False
      ),
      out_type=jax.ShapeDtypeStruct((_ROWS * _COLS, _DP), jnp.float32),
      scratch_types=[
          pltpu.VMEM((_COLS_PAD, _ROWS_W), jnp.int32),
          pltpu.VMEM((_ROWS_W, _COLS_PAD), jnp.int32),
          [pltpu.VMEM((_COLS, _D), jnp.float32) for _ in range(_NBUF)],
          pltpu.SemaphoreType.DMA,
          [pltpu.SemaphoreType.DMA for _ in range(_NBUF)],
          [pltpu.SemaphoreType.DMA for _ in range(_NBUF)],
      ],
  )
  def emb_kernel(tok_hbm, tab_hbm, out_hbm, blk, idxt, rows, isem, gsems,
                 ssems):
    wid = lax.axis_index("s") * _NC + lax.axis_index("c")
    row0 = wid * _ROWS_W

    blk_cp = pltpu.make_async_copy(
        tok_hbm.at[:, pl.ds(row0, _ROWS_W)], blk.at[pl.ds(0, _COLS)], isem
    )
    blk_cp.start()
    blk_cp.wait()

    lanes = lax.iota(jnp.int32, 16)
    cvecs = [lanes + 16 * cb for cb in range(_COLS_PAD // 16)]

    def trans_body(i, carry):
      r0 = (i // 16) * 16
      d = i % 16
      m = (d + lanes) % 16
      rvec = r0 + m
      for cvec in cvecs:
        v = plsc.load_gather(blk, [cvec, rvec])
        plsc.store_scatter(idxt, [rvec, cvec], v)
      return carry

    lax.fori_loop(0, _ROWS_W, trans_body, 0)

    def gathers(r, b):
      return [
          pltpu.make_async_copy(
              tab_hbm.at[idxt.at[r, pl.ds(off, sz)]],
              rows[b].at[pl.ds(off, sz)],
              gsems[b],
          )
          for off, sz in _SPLITS
      ]

    def store(r, b):
      return pltpu.make_async_copy(
          rows[b],
          out_hbm.at[pl.ds((row0 + r) * _COLS, _COLS), pl.ds(0, _D)],
          ssems[b],
      )

    for b in range(_LOOK):
      for g in gathers(b, b):
        g.start()

    def body(i, carry):
      for b in range(_NBUF):
        r = i * _NBUF + b
        for g in gathers(r, b):
          g.wait()
        store(r, b).start()

        t = r + _LOOK
        bt = (b + _LOOK) % _NBUF

        @pl.when(t < _ROWS_W)
        def _():
          @pl.when(t >= _NBUF)
          def _():
            store(t - _NBUF, bt).wait()

          for g in gathers(t, bt):
            g.start()

      return carry

    lax.fori_loop(0, _STEPS, body, 0)
    for b in range(_NBUF):
      store(_ROWS_W - _NBUF + b, b).wait()

  return emb_kernel(tokens_t, table)


def kernel(tokens, embedding):
  out2 = _sc_embedding_gather(tokens.T.astype(jnp.int32), embedding)
  return out2[:, :_D].reshape(_ROWS, _COLS, _D)
